# baseline (speedup 1.0000x reference)
import jax
import jax.numpy as jnp
from jax import lax
from jax.experimental import pallas as pl
from jax.experimental.pallas import tpu as pltpu
from jax.experimental.pallas import tpu_sc as plsc

_TABLE_SIZE = 524288
_MASK = _TABLE_SIZE - 1
_NWORDS = _TABLE_SIZE // 8
_N = 1048576
_P32 = -1640531535
_NC, _NS = 2, 16
_NW = _NC * _NS

_PACK_PER_TILE = _NWORDS // _NW
_PACK_CHUNKS = 16


def _pack_body(t_xy, t_xz, t_yz, p_xy, p_xz, p_yz, rb0, rb1, wbuf, sr0, sr1):
    i32 = jnp.int32
    wid = lax.axis_index("s") * i32(_NC) + lax.axis_index("c")
    iota = lax.iota(jnp.int32, 16)
    iota8 = iota * i32(8)
    rbufs = (rb0, rb1)
    rsems = (sr0, sr1)
    for t_ref, p_ref in ((t_xy, p_xy), (t_xz, p_xz), (t_yz, p_yz)):
        word0 = wid * i32(_PACK_PER_TILE)

        def t_slice(ci, t_ref=t_ref):
            return t_ref.at[pl.ds((word0 + ci * i32(128)) * i32(32), 4096)]

        pltpu.async_copy(t_slice(i32(0)), rbufs[0], rsems[0])

        def pair_body(pi, carry, t_ref=t_ref):
            for b in (0, 1):
                ci = pi * i32(2) + i32(b)
                nb = b ^ 1

                @pl.when(ci < i32(_PACK_CHUNKS - 1))
                def _(ci=ci, nb=nb, t_ref=t_ref):
                    pltpu.async_copy(t_slice(ci + i32(1)), rbufs[nb], rsems[nb])

                pltpu.make_async_copy(t_slice(ci), rbufs[b], rsems[b]).wait()
                rowbuf = rbufs[b]

                def group_body(g, carry2, rowbuf=rowbuf):
                    zero = jnp.zeros((16,), jnp.int32)
                    parts = [zero, zero, zero, zero]
                    gb = g * i32(512)
                    for j in range(32):
                        idx = iota8 + (gb + i32(((j & 3) << 7) + (j >> 2)))
                        v = plsc.load_gather(rowbuf, [idx])
                        bit = jnp.where(
                            v >= 0.0,
                            i32(1 << j if j < 31 else -(1 << 31)),
                            i32(0),
                        )
                        parts[j & 3] = parts[j & 3] | bit
                    w = (parts[0] | parts[1]) | (parts[2] | parts[3])
                    wbuf[pl.ds(ci * i32(128) + g * i32(16), 16)] = w
                    return carry2

                lax.fori_loop(i32(0), i32(8), group_body, i32(0))
            return carry

        lax.fori_loop(i32(0), i32(_PACK_CHUNKS // 2), pair_body, i32(0))
        pltpu.sync_copy(wbuf, p_ref.at[pl.ds(word0, _PACK_PER_TILE)])


_POS_PER_TILE = _N // _NW
_CHUNK = 2048
_NCHUNKS = _POS_PER_TILE // _CHUNK
_NGROUPS = _CHUNK // 16


def _main_body(pos, p_xy, p_xz, p_yz, out, tbl, pb0, pb1, or0, or1, ow0, ow1,
               lut, sp0, sp1, so0, so1, sw0, sw1):
    i32 = jnp.int32
    f32 = jnp.float32
    wid = lax.axis_index("s") * i32(_NC) + lax.axis_index("c")
    iota = lax.iota(jnp.int32, 16)
    for f in range(4):
        bits = (iota >> i32(f)) & i32(1)
        lut[pl.ds(f * 16, 16)] = (i32(2) * bits - i32(1)).astype(jnp.float32)
    pos0 = wid * i32(_POS_PER_TILE)
    pbufs, orbufs, owbufs = (pb0, pb1), (or0, or1), (ow0, ow1)
    psems, osems, wsems = (sp0, sp1), (so0, so1), (sw0, sw1)

    def pos_slice(ci):
        return pos.at[pl.ds((pos0 + ci * i32(_CHUNK)) * i32(4), _CHUNK * 4)]

    def out_slice(ci):
        return out.at[pl.ds((pos0 + ci * i32(_CHUNK)) * i32(4), _CHUNK * 4)]

    for plane, (p_ref, ca, cb) in enumerate(
        ((p_xy, 0, 1), (p_xz, 0, 2), (p_yz, 1, 2))
    ):
        pltpu.sync_copy(p_ref, tbl)
        pltpu.async_copy(pos_slice(i32(0)), pbufs[0], psems[0])
        if plane > 0:
            pltpu.async_copy(out_slice(i32(0)), orbufs[0], osems[0])

        def pair_body(pi, carry, plane=plane, ca=ca, cb=cb):
            for b in (0, 1):
                ci = pi * i32(2) + i32(b)
                nb = b ^ 1

                @pl.when(ci < i32(_NCHUNKS - 1))
                def _(b=b, nb=nb, ci=ci, plane=plane):
                    pltpu.async_copy(pos_slice(ci + i32(1)), pbufs[nb], psems[nb])
                    if plane > 0:
                        pltpu.async_copy(
                            out_slice(ci + i32(1)), orbufs[nb], osems[nb]
                        )

                pltpu.make_async_copy(pos_slice(ci), pbufs[b], psems[b]).wait()
                if plane > 0:
                    pltpu.make_async_copy(out_slice(ci), orbufs[b], osems[b]).wait()

                @pl.when(ci >= i32(2))
                def _(b=b, ci=ci):
                    pltpu.make_async_copy(owbufs[b], out_slice(ci), wsems[b]).wait()

                posb, orb, owb = pbufs[b], orbufs[b], owbufs[b]

                @plsc.parallel_loop(i32(0), i32(_NGROUPS), i32(1), unroll=4)
                def group_body(g, posb=posb, orb=orb, owb=owb,
                               plane=plane, ca=ca, cb=cb):
                    goff = (g >> i32(3)) * i32(512) + (g & i32(7)) * i32(16)
                    a = posb[pl.ds(goff + i32(ca * 128), 16)]
                    b2 = posb[pl.ds(goff + i32(cb * 128), 16)]
                    sa = a * f32(1024.0)
                    sb = b2 * f32(1024.0)
                    ia = sa.astype(jnp.int32)
                    ib = sb.astype(jnp.int32)
                    wa = sa - ia.astype(jnp.float32)
                    wb = sb - ib.astype(jnp.float32)
                    ua = f32(1.0) - wa
                    ub = f32(1.0) - wb
                    g0 = ib * i32(_P32)
                    g1 = g0 + i32(_P32)
                    ia1 = ia + i32(1)
                    corners = (
                        (ia ^ g0, ua * ub),
                        (ia ^ g1, ua * wb),
                        (ia1 ^ g0, wa * ub),
                        (ia1 ^ g1, wa * wb),
                    )
                    acc = [None] * 4
                    for hc, wc in corners:
                        hm = hc & i32(_MASK)
                        wi = hm >> i32(3)
                        sh = (hm & i32(7)) << i32(2)
                        word = plsc.load_gather(tbl, [wi])
                        nib = (word >> sh) & i32(15)
                        for f in range(4):
                            s = plsc.load_gather(lut, [nib + i32(f * 16)])
                            t = wc * s
                            acc[f] = t if acc[f] is None else acc[f] + t
                    for f in range(4):
                        off = goff + i32(f * 128)
                        if plane == 0:
                            owb[pl.ds(off, 16)] = acc[f]
                        else:
                            owb[pl.ds(off, 16)] = orb[pl.ds(off, 16)] + acc[f]

                pltpu.async_copy(owbufs[b], out_slice(ci), wsems[b])
            return carry

        lax.fori_loop(i32(0), i32(_NCHUNKS // 2), pair_body, i32(0))
        pltpu.make_async_copy(
            owbufs[0], out_slice(i32(_NCHUNKS - 2)), wsems[0]
        ).wait()
        pltpu.make_async_copy(
            owbufs[1], out_slice(i32(_NCHUNKS - 1)), wsems[1]
        ).wait()


def _to_blocked(t):
    r = t.shape[0]
    return t.reshape(r // 128, 128, 4).transpose(0, 2, 1).reshape(-1)


def kernel(positions, table_xy, table_xz, table_yz):
    mesh = plsc.VectorSubcoreMesh(core_axis_name="c", subcore_axis_name="s")
    params = pltpu.CompilerParams(needs_layout_passes=False)
    pack = pl.kernel(
        _pack_body,
        out_type=(jax.ShapeDtypeStruct((_NWORDS,), jnp.int32),) * 3,
        mesh=mesh,
        compiler_params=params,
        scratch_types=[
            pltpu.VMEM((4096,), jnp.float32),
            pltpu.VMEM((4096,), jnp.float32),
            pltpu.VMEM((_PACK_PER_TILE,), jnp.int32),
            pltpu.SemaphoreType.DMA,
            pltpu.SemaphoreType.DMA,
        ],
    )
    p_xy, p_xz, p_yz = pack(
        _to_blocked(table_xy), _to_blocked(table_xz), _to_blocked(table_yz)
    )
    main = pl.kernel(
        _main_body,
        out_type=jax.ShapeDtypeStruct((_N * 4,), jnp.float32),
        mesh=mesh,
        compiler_params=params,
        scratch_types=[
            pltpu.VMEM((_NWORDS,), jnp.int32),
            pltpu.VMEM((_CHUNK * 4,), jnp.float32),
            pltpu.VMEM((_CHUNK * 4,), jnp.float32),
            pltpu.VMEM((_CHUNK * 4,), jnp.float32),
            pltpu.VMEM((_CHUNK * 4,), jnp.float32),
            pltpu.VMEM((_CHUNK * 4,), jnp.float32),
            pltpu.VMEM((_CHUNK * 4,), jnp.float32),
            pltpu.VMEM((64,), jnp.float32),
            pltpu.SemaphoreType.DMA,
            pltpu.SemaphoreType.DMA,
            pltpu.SemaphoreType.DMA,
            pltpu.SemaphoreType.DMA,
            pltpu.SemaphoreType.DMA,
            pltpu.SemaphoreType.DMA,
        ],
    )
    pos4 = jnp.pad(positions, ((0, 0), (0, 1)))
    flat = main(_to_blocked(pos4), p_xy, p_xz, p_yz)
    return (
        flat.reshape(_N // 128, 4, 128).transpose(0, 2, 1).reshape(_N, 4)
    )


# speedup vs baseline: 1.1053x; 1.1053x over previous
import jax
import jax.numpy as jnp
from jax import lax
from jax.experimental import pallas as pl
from jax.experimental.pallas import tpu as pltpu
from jax.experimental.pallas import tpu_sc as plsc

_TABLE_SIZE = 524288
_MASK = _TABLE_SIZE - 1
_NWORDS = _TABLE_SIZE // 8
_N = 1048576
_P32 = -1640531535
_NC, _NS = 2, 16
_NW = _NC * _NS

_PACK_PER_TILE = _NWORDS // _NW
_PACK_CHUNKS = 16


def _pack_body(t_xy, t_xz, t_yz, p_xy, p_xz, p_yz, rb0, rb1, wbuf, sr0, sr1):
    i32 = jnp.int32
    wid = lax.axis_index("s") * i32(_NC) + lax.axis_index("c")
    iota = lax.iota(jnp.int32, 16)
    iota8 = iota * i32(8)
    rbufs = (rb0, rb1)
    rsems = (sr0, sr1)
    for t_ref, p_ref in ((t_xy, p_xy), (t_xz, p_xz), (t_yz, p_yz)):
        word0 = wid * i32(_PACK_PER_TILE)

        def t_slice(ci, t_ref=t_ref):
            return t_ref.at[pl.ds((word0 + ci * i32(128)) * i32(32), 4096)]

        pltpu.async_copy(t_slice(i32(0)), rbufs[0], rsems[0])

        def pair_body(pi, carry, t_ref=t_ref):
            for b in (0, 1):
                ci = pi * i32(2) + i32(b)
                nb = b ^ 1

                @pl.when(ci < i32(_PACK_CHUNKS - 1))
                def _(ci=ci, nb=nb, t_ref=t_ref):
                    pltpu.async_copy(t_slice(ci + i32(1)), rbufs[nb], rsems[nb])

                pltpu.make_async_copy(t_slice(ci), rbufs[b], rsems[b]).wait()
                rowbuf = rbufs[b]

                @plsc.parallel_loop(i32(0), i32(8), i32(1), unroll=2)
                def group_body(g, rowbuf=rowbuf):
                    zero = jnp.zeros((16,), jnp.int32)
                    parts = [zero, zero, zero, zero]
                    gb = g * i32(512)
                    for j in range(32):
                        idx = iota8 + (gb + i32(((j & 3) << 7) + (j >> 2)))
                        v = plsc.load_gather(rowbuf, [idx])
                        bit = jnp.where(
                            v >= 0.0,
                            i32(1 << j if j < 31 else -(1 << 31)),
                            i32(0),
                        )
                        parts[j & 3] = parts[j & 3] | bit
                    w = (parts[0] | parts[1]) | (parts[2] | parts[3])
                    wbuf[pl.ds(ci * i32(128) + g * i32(16), 16)] = w
            return carry

        lax.fori_loop(i32(0), i32(_PACK_CHUNKS // 2), pair_body, i32(0))
        pltpu.sync_copy(wbuf, p_ref.at[pl.ds(word0, _PACK_PER_TILE)])


_POS_PER_TILE = _N // _NW
_CHUNK = 2048
_NCHUNKS = _POS_PER_TILE // _CHUNK
_NGROUPS = _CHUNK // 16


def _main_body(pos, p_xy, p_xz, p_yz, out, tbl, pb0, pb1, or0, or1, ow0, ow1,
               lut, sp0, sp1, so0, so1, sw0, sw1):
    i32 = jnp.int32
    f32 = jnp.float32
    wid = lax.axis_index("s") * i32(_NC) + lax.axis_index("c")
    iota = lax.iota(jnp.int32, 16)
    for f in range(4):
        bits = (iota >> i32(f)) & i32(1)
        lut[pl.ds(f * 16, 16)] = (i32(2) * bits - i32(1)).astype(jnp.float32)
    pos0 = wid * i32(_POS_PER_TILE)
    pbufs, orbufs, owbufs = (pb0, pb1), (or0, or1), (ow0, ow1)
    psems, osems, wsems = (sp0, sp1), (so0, so1), (sw0, sw1)

    def pos_slice(ci):
        return pos.at[pl.ds((pos0 + ci * i32(_CHUNK)) * i32(4), _CHUNK * 4)]

    def out_slice(ci):
        return out.at[pl.ds((pos0 + ci * i32(_CHUNK)) * i32(4), _CHUNK * 4)]

    for plane, (p_ref, ca, cb) in enumerate(
        ((p_xy, 0, 1), (p_xz, 0, 2), (p_yz, 1, 2))
    ):
        pltpu.sync_copy(p_ref, tbl)
        pltpu.async_copy(pos_slice(i32(0)), pbufs[0], psems[0])
        if plane > 0:
            pltpu.async_copy(out_slice(i32(0)), orbufs[0], osems[0])

        def pair_body(pi, carry, plane=plane, ca=ca, cb=cb):
            for b in (0, 1):
                ci = pi * i32(2) + i32(b)
                nb = b ^ 1

                @pl.when(ci < i32(_NCHUNKS - 1))
                def _(b=b, nb=nb, ci=ci, plane=plane):
                    pltpu.async_copy(pos_slice(ci + i32(1)), pbufs[nb], psems[nb])
                    if plane > 0:
                        pltpu.async_copy(
                            out_slice(ci + i32(1)), orbufs[nb], osems[nb]
                        )

                pltpu.make_async_copy(pos_slice(ci), pbufs[b], psems[b]).wait()
                if plane > 0:
                    pltpu.make_async_copy(out_slice(ci), orbufs[b], osems[b]).wait()

                @pl.when(ci >= i32(2))
                def _(b=b, ci=ci):
                    pltpu.make_async_copy(owbufs[b], out_slice(ci), wsems[b]).wait()

                posb, orb, owb = pbufs[b], orbufs[b], owbufs[b]

                @plsc.parallel_loop(i32(0), i32(_NGROUPS), i32(1), unroll=2)
                def group_body(g, posb=posb, orb=orb, owb=owb,
                               plane=plane, ca=ca, cb=cb):
                    goff = (g >> i32(3)) * i32(512) + (g & i32(7)) * i32(16)
                    a = posb[pl.ds(goff + i32(ca * 128), 16)]
                    b2 = posb[pl.ds(goff + i32(cb * 128), 16)]
                    sa = a * f32(1024.0)
                    sb = b2 * f32(1024.0)
                    ia = sa.astype(jnp.int32)
                    ib = sb.astype(jnp.int32)
                    wa = sa - ia.astype(jnp.float32)
                    wb = sb - ib.astype(jnp.float32)
                    ua = f32(1.0) - wa
                    ub = f32(1.0) - wb
                    g0 = ib * i32(_P32)
                    g1 = g0 + i32(_P32)
                    ia1 = ia + i32(1)
                    corners = (
                        (ia ^ g0, ua * ub),
                        (ia ^ g1, ua * wb),
                        (ia1 ^ g0, wa * ub),
                        (ia1 ^ g1, wa * wb),
                    )
                    acc = [None] * 4
                    for hc, wc in corners:
                        hm = hc & i32(_MASK)
                        wi = hm >> i32(3)
                        sh = (hm & i32(7)) << i32(2)
                        word = plsc.load_gather(tbl, [wi])
                        nib = (word >> sh) & i32(15)
                        for f in range(4):
                            s = plsc.load_gather(lut, [nib + i32(f * 16)])
                            t = wc * s
                            acc[f] = t if acc[f] is None else acc[f] + t
                    for f in range(4):
                        off = goff + i32(f * 128)
                        if plane == 0:
                            owb[pl.ds(off, 16)] = acc[f]
                        else:
                            owb[pl.ds(off, 16)] = orb[pl.ds(off, 16)] + acc[f]

                pltpu.async_copy(owbufs[b], out_slice(ci), wsems[b])
            return carry

        lax.fori_loop(i32(0), i32(_NCHUNKS // 2), pair_body, i32(0))
        pltpu.make_async_copy(
            owbufs[0], out_slice(i32(_NCHUNKS - 2)), wsems[0]
        ).wait()
        pltpu.make_async_copy(
            owbufs[1], out_slice(i32(_NCHUNKS - 1)), wsems[1]
        ).wait()


def _to_blocked(t):
    r = t.shape[0]
    return t.reshape(r // 128, 128, 4).transpose(0, 2, 1).reshape(-1)


def kernel(positions, table_xy, table_xz, table_yz):
    mesh = plsc.VectorSubcoreMesh(core_axis_name="c", subcore_axis_name="s")
    params = pltpu.CompilerParams(needs_layout_passes=False)
    pack = pl.kernel(
        _pack_body,
        out_type=(jax.ShapeDtypeStruct((_NWORDS,), jnp.int32),) * 3,
        mesh=mesh,
        compiler_params=params,
        scratch_types=[
            pltpu.VMEM((4096,), jnp.float32),
            pltpu.VMEM((4096,), jnp.float32),
            pltpu.VMEM((_PACK_PER_TILE,), jnp.int32),
            pltpu.SemaphoreType.DMA,
            pltpu.SemaphoreType.DMA,
        ],
    )
    p_xy, p_xz, p_yz = pack(
        _to_blocked(table_xy), _to_blocked(table_xz), _to_blocked(table_yz)
    )
    main = pl.kernel(
        _main_body,
        out_type=jax.ShapeDtypeStruct((_N * 4,), jnp.float32),
        mesh=mesh,
        compiler_params=params,
        scratch_types=[
            pltpu.VMEM((_NWORDS,), jnp.int32),
            pltpu.VMEM((_CHUNK * 4,), jnp.float32),
            pltpu.VMEM((_CHUNK * 4,), jnp.float32),
            pltpu.VMEM((_CHUNK * 4,), jnp.float32),
            pltpu.VMEM((_CHUNK * 4,), jnp.float32),
            pltpu.VMEM((_CHUNK * 4,), jnp.float32),
            pltpu.VMEM((_CHUNK * 4,), jnp.float32),
            pltpu.VMEM((64,), jnp.float32),
            pltpu.SemaphoreType.DMA,
            pltpu.SemaphoreType.DMA,
            pltpu.SemaphoreType.DMA,
            pltpu.SemaphoreType.DMA,
            pltpu.SemaphoreType.DMA,
            pltpu.SemaphoreType.DMA,
        ],
    )
    pos4 = jnp.pad(positions, ((0, 0), (0, 1)))
    flat = main(_to_blocked(pos4), p_xy, p_xz, p_yz)
    return (
        flat.reshape(_N // 128, 4, 128).transpose(0, 2, 1).reshape(_N, 4)
    )
